# baseline (speedup 1.0000x reference)
import jax
import jax.numpy as jnp
from jax.experimental import pallas as pl
from jax.experimental.pallas import tpu as pltpu
from jax.experimental.pallas import tpu_sc as plsc

D_MODEL = 128
SCALE = float(D_MODEL) ** 0.5
GATHER_WINDOW = 100


def _scale_table(table):
    rows = table.shape[0]
    block_rows = 10000
    grid = rows // block_rows

    def body(t_ref, o_ref):
        o_ref[...] = t_ref[...] * SCALE

    return pl.pallas_call(
        body,
        grid=(grid,),
        in_specs=[pl.BlockSpec((block_rows, D_MODEL), lambda i: (i, 0))],
        out_specs=pl.BlockSpec((block_rows, D_MODEL), lambda i: (i, 0)),
        out_shape=jax.ShapeDtypeStruct(table.shape, table.dtype),
    )(table)


N_STREAMS = 4


def _sc_gather(table, indices):
    num_indices = indices.shape[1]
    rows_per_step = N_STREAMS * GATHER_WINDOW
    mesh = plsc.VectorSubcoreMesh(core_axis_name="c", subcore_axis_name="s")
    idx2d = indices.reshape(num_indices // GATHER_WINDOW, GATHER_WINDOW)

    @pl.kernel(
        out_type=jax.ShapeDtypeStruct((num_indices, D_MODEL), table.dtype),
        mesh=mesh,
        scratch_types=[pltpu.SemaphoreType.DMA],
    )
    def k(table_hbm, idx_hbm, out_hbm, sem):
        def body(idx_vmem, out_vmem):
            copies = [
                pltpu.async_copy(
                    table_hbm.at[idx_vmem.at[j]],
                    out_vmem.at[pl.ds(j * GATHER_WINDOW, GATHER_WINDOW)],
                    sem,
                )
                for j in range(N_STREAMS)
            ]
            for c in copies:
                c.wait()

        pltpu.emit_pipeline(
            body,
            grid=(num_indices // rows_per_step,),
            in_specs=[
                pl.BlockSpec(
                    (N_STREAMS, GATHER_WINDOW), index_map=lambda i: (i, 0)
                )
            ],
            out_specs=[
                pl.BlockSpec((rows_per_step, D_MODEL), index_map=lambda i: (i, 0))
            ],
            core_axis_name=("c", "s"),
            dimension_semantics=(pltpu.PARALLEL,),
        )(idx_hbm, out_hbm)

    return k(table, idx2d)


def kernel(x, emb_weight):
    batch, seq = x.shape
    idx_t = x.astype(jnp.int32).T.reshape(1, -1)
    flat = _sc_gather(_scale_table(emb_weight), idx_t)
    out_t = flat.reshape(seq, batch, D_MODEL)
    return jnp.swapaxes(out_t, 0, 1)


# speedup vs baseline: 1.0065x; 1.0065x over previous
import jax
import jax.numpy as jnp
from jax.experimental import pallas as pl
from jax.experimental.pallas import tpu as pltpu
from jax.experimental.pallas import tpu_sc as plsc

D_MODEL = 128
SCALE = float(D_MODEL) ** 0.5
GATHER_WINDOW = 64


def _scale_table(table):
    rows = table.shape[0]
    block_rows = 10000
    grid = rows // block_rows

    def body(t_ref, o_ref):
        o_ref[...] = t_ref[...] * SCALE

    return pl.pallas_call(
        body,
        grid=(grid,),
        in_specs=[pl.BlockSpec((block_rows, D_MODEL), lambda i: (i, 0))],
        out_specs=pl.BlockSpec((block_rows, D_MODEL), lambda i: (i, 0)),
        out_shape=jax.ShapeDtypeStruct(table.shape, table.dtype),
    )(table)


N_STREAMS = 4


def _sc_gather(table, indices):
    num_indices = indices.shape[1]
    rows_per_step = N_STREAMS * GATHER_WINDOW
    mesh = plsc.VectorSubcoreMesh(core_axis_name="c", subcore_axis_name="s")
    idx2d = indices.reshape(num_indices // GATHER_WINDOW, GATHER_WINDOW)

    @pl.kernel(
        out_type=jax.ShapeDtypeStruct((num_indices, D_MODEL), table.dtype),
        mesh=mesh,
        scratch_types=[pltpu.SemaphoreType.DMA],
    )
    def k(table_hbm, idx_hbm, out_hbm, sem):
        def body(idx_vmem, out_vmem):
            copies = [
                pltpu.async_copy(
                    table_hbm.at[idx_vmem.at[j]],
                    out_vmem.at[pl.ds(j * GATHER_WINDOW, GATHER_WINDOW)],
                    sem,
                )
                for j in range(N_STREAMS)
            ]
            for c in copies:
                c.wait()

        pltpu.emit_pipeline(
            body,
            grid=(num_indices // rows_per_step,),
            in_specs=[
                pl.BlockSpec(
                    (N_STREAMS, GATHER_WINDOW), index_map=lambda i: (i, 0)
                )
            ],
            out_specs=[
                pl.BlockSpec((rows_per_step, D_MODEL), index_map=lambda i: (i, 0))
            ],
            core_axis_name=("c", "s"),
            dimension_semantics=(pltpu.PARALLEL,),
        )(idx_hbm, out_hbm)

    return k(table, idx2d)


def kernel(x, emb_weight):
    batch, seq = x.shape
    idx_t = x.astype(jnp.int32).T.reshape(1, -1)
    flat = _sc_gather(_scale_table(emb_weight), idx_t)
    out_t = flat.reshape(seq, batch, D_MODEL)
    return jnp.swapaxes(out_t, 0, 1)


# speedup vs baseline: 1.0143x; 1.0078x over previous
import jax
import jax.numpy as jnp
from jax.experimental import pallas as pl
from jax.experimental.pallas import tpu as pltpu
from jax.experimental.pallas import tpu_sc as plsc

D_MODEL = 128
SCALE = float(D_MODEL) ** 0.5
GATHER_WINDOW = 128


def _scale_table(table):
    rows = table.shape[0]
    block_rows = 10000
    grid = rows // block_rows

    def body(t_ref, o_ref):
        o_ref[...] = t_ref[...] * SCALE

    return pl.pallas_call(
        body,
        grid=(grid,),
        in_specs=[pl.BlockSpec((block_rows, D_MODEL), lambda i: (i, 0))],
        out_specs=pl.BlockSpec((block_rows, D_MODEL), lambda i: (i, 0)),
        out_shape=jax.ShapeDtypeStruct(table.shape, table.dtype),
        compiler_params=pltpu.CompilerParams(
            dimension_semantics=("parallel",)
        ),
    )(table)


N_STREAMS = 2


def _sc_gather(table, indices):
    num_indices = indices.shape[1]
    rows_per_step = N_STREAMS * GATHER_WINDOW
    mesh = plsc.VectorSubcoreMesh(core_axis_name="c", subcore_axis_name="s")
    idx2d = indices.reshape(num_indices // GATHER_WINDOW, GATHER_WINDOW)

    @pl.kernel(
        out_type=jax.ShapeDtypeStruct((num_indices, D_MODEL), table.dtype),
        mesh=mesh,
        scratch_types=[pltpu.SemaphoreType.DMA],
    )
    def k(table_hbm, idx_hbm, out_hbm, sem):
        def body(idx_vmem, out_vmem):
            copies = [
                pltpu.async_copy(
                    table_hbm.at[idx_vmem.at[j]],
                    out_vmem.at[pl.ds(j * GATHER_WINDOW, GATHER_WINDOW)],
                    sem,
                )
                for j in range(N_STREAMS)
            ]
            for c in copies:
                c.wait()

        pltpu.emit_pipeline(
            body,
            grid=(num_indices // rows_per_step,),
            in_specs=[
                pl.BlockSpec(
                    (N_STREAMS, GATHER_WINDOW), index_map=lambda i: (i, 0)
                )
            ],
            out_specs=[
                pl.BlockSpec((rows_per_step, D_MODEL), index_map=lambda i: (i, 0))
            ],
            core_axis_name=("c", "s"),
            dimension_semantics=(pltpu.PARALLEL,),
        )(idx_hbm, out_hbm)

    return k(table, idx2d)


def kernel(x, emb_weight):
    batch, seq = x.shape
    idx_t = x.astype(jnp.int32).T.reshape(1, -1)
    flat = _sc_gather(_scale_table(emb_weight), idx_t)
    out_t = flat.reshape(seq, batch, D_MODEL)
    return jnp.swapaxes(out_t, 0, 1)
